# Initial kernel scaffold; baseline (speedup 1.0000x reference)
import jax
import jax.numpy as jnp
from jax.experimental import pallas as pl


def kernel(features, labels, centers):
    raise NotImplementedError("write your pallas kernel here")


# speedup vs baseline: 1.6187x; 1.6187x over previous
import jax
import jax.numpy as jnp
from jax.experimental import pallas as pl

_BB = 512
_CPAD = 1024


def _block_kernel(lab_ref, f_ref, c_ref, out_ref):
    i = pl.program_id(0)
    lab = lab_ref[0]
    col = jax.lax.broadcasted_iota(jnp.int32, (_BB, _CPAD), 1)
    onehot = (col == lab).astype(jnp.float32)
    bc = jnp.dot(onehot, c_ref[...], preferred_element_type=jnp.float32)
    d = f_ref[...] - bc
    part = jnp.sum(d * d, keepdims=True)

    @pl.when(i == 0)
    def _init():
        out_ref[...] = jnp.zeros((1, 1), jnp.float32)

    out_ref[...] += part


def kernel(features, labels, centers):
    batch, feat = features.shape
    nclass = centers.shape[0]
    g = batch // _BB
    lab3 = labels.astype(jnp.int32).reshape(g, _BB, 1)
    cpad = jnp.pad(centers, ((0, _CPAD - nclass), (0, 0)))
    total = pl.pallas_call(
        _block_kernel,
        grid=(g,),
        in_specs=[
            pl.BlockSpec((1, _BB, 1), lambda i: (i, 0, 0)),
            pl.BlockSpec((_BB, feat), lambda i: (i, 0)),
            pl.BlockSpec((_CPAD, feat), lambda i: (0, 0)),
        ],
        out_specs=pl.BlockSpec((1, 1), lambda i: (0, 0)),
        out_shape=jax.ShapeDtypeStruct((1, 1), jnp.float32),
    )(lab3, features, cpad)
    return (0.5 / batch) * total[0, 0]


# speedup vs baseline: 1.6628x; 1.0272x over previous
import jax
import jax.numpy as jnp
from jax.experimental import pallas as pl

_BB = 512
_CPAD = 1024


def _block_kernel(lab_ref, f_ref, c_ref, out_ref):
    i = pl.program_id(0)
    lab = lab_ref[0]
    col = jax.lax.broadcasted_iota(jnp.int32, (_BB, _CPAD), 1)
    onehot = (col == lab).astype(jnp.bfloat16)
    bc = jnp.dot(onehot, c_ref[...], preferred_element_type=jnp.float32)
    d = f_ref[...] - bc
    part = jnp.sum(d * d, keepdims=True)

    @pl.when(i == 0)
    def _init():
        out_ref[...] = jnp.zeros((1, 1), jnp.float32)

    out_ref[...] += part


def kernel(features, labels, centers):
    batch, feat = features.shape
    nclass = centers.shape[0]
    g = batch // _BB
    lab3 = labels.astype(jnp.int32).reshape(g, _BB, 1)
    cpad = jnp.pad(centers, ((0, _CPAD - nclass), (0, 0))).astype(jnp.bfloat16)
    total = pl.pallas_call(
        _block_kernel,
        grid=(g,),
        in_specs=[
            pl.BlockSpec((1, _BB, 1), lambda i: (i, 0, 0)),
            pl.BlockSpec((_BB, feat), lambda i: (i, 0)),
            pl.BlockSpec((_CPAD, feat), lambda i: (0, 0)),
        ],
        out_specs=pl.BlockSpec((1, 1), lambda i: (0, 0)),
        out_shape=jax.ShapeDtypeStruct((1, 1), jnp.float32),
    )(lab3, features, cpad)
    return (0.5 / batch) * total[0, 0]


# speedup vs baseline: 1.7288x; 1.0397x over previous
import jax
import jax.numpy as jnp
from jax.experimental import pallas as pl

_BB = 1024
_CPAD = 1024


def _block_kernel(lab_ref, f_ref, c_ref, out_ref):
    i = pl.program_id(0)
    lab = lab_ref[0]
    col = jax.lax.broadcasted_iota(jnp.int32, (_BB, _CPAD), 1)
    onehot = (col == lab).astype(jnp.bfloat16)
    bc = jnp.dot(onehot, c_ref[...], preferred_element_type=jnp.float32)
    d = f_ref[...] - bc
    part = jnp.sum(d * d, keepdims=True)

    @pl.when(i == 0)
    def _init():
        out_ref[...] = jnp.zeros((1, 1), jnp.float32)

    out_ref[...] += part


def kernel(features, labels, centers):
    batch, feat = features.shape
    nclass = centers.shape[0]
    g = batch // _BB
    lab3 = labels.astype(jnp.int32).reshape(g, _BB, 1)
    cpad = jnp.pad(centers, ((0, _CPAD - nclass), (0, 0))).astype(jnp.bfloat16)
    total = pl.pallas_call(
        _block_kernel,
        grid=(g,),
        in_specs=[
            pl.BlockSpec((1, _BB, 1), lambda i: (i, 0, 0)),
            pl.BlockSpec((_BB, feat), lambda i: (i, 0)),
            pl.BlockSpec((_CPAD, feat), lambda i: (0, 0)),
        ],
        out_specs=pl.BlockSpec((1, 1), lambda i: (0, 0)),
        out_shape=jax.ShapeDtypeStruct((1, 1), jnp.float32),
    )(lab3, features, cpad)
    return (0.5 / batch) * total[0, 0]
